# Initial kernel scaffold; baseline (speedup 1.0000x reference)
import jax
import jax.numpy as jnp
from jax.experimental import pallas as pl


def kernel(x, edge_index, W, b):
    raise NotImplementedError("write your pallas kernel here")


# speedup vs baseline: 16.8601x; 16.8601x over previous
import functools

import jax
import jax.numpy as jnp
from jax import lax
from jax.experimental import pallas as pl
from jax.experimental.pallas import tpu as pltpu
from jax.experimental.pallas import tpu_sc as plsc

NC = 2
NS = 16
NW = NC * NS
K = 128
DEGW = 16


def _mesh():
    return plsc.VectorSubcoreMesh(
        core_axis_name="c", subcore_axis_name="s", num_cores=NC, num_subcores=NS
    )


def _fill_rows(ref, n_rows, n_lane_groups, value):
    vec = jnp.full((16,), value, jnp.float32)

    def body(i, _):
        for j in range(n_lane_groups):
            ref[i, pl.ds(16 * j, 16)] = vec
        return 0

    lax.fori_loop(0, n_rows, body, 0)


def _deg_call(dst_e, npad, chunks):
    rt = npad // NS

    @functools.partial(
        pl.kernel,
        out_type=jax.ShapeDtypeStruct((NC * npad, DEGW), jnp.float32),
        mesh=_mesh(),
        scratch_types=[
            pltpu.VMEM_SHARED((npad, DEGW), jnp.float32),
            pltpu.VMEM((K,), jnp.int32),
            pltpu.VMEM((K, DEGW), jnp.float32),
            pltpu.VMEM((rt, DEGW), jnp.float32),
        ],
    )
    def deg_k(dst_hbm, degpart_hbm, deg_sh, idx_v, ones_v, z_v):
        c = lax.axis_index("c")
        s = lax.axis_index("s")
        w = c * NS + s
        _fill_rows(z_v, rt, DEGW // 16, 0.0)
        pltpu.sync_copy(z_v, deg_sh.at[pl.ds(s * rt, rt)])
        _fill_rows(ones_v, K, DEGW // 16, 1.0)
        plsc.subcore_barrier()

        base = w * chunks * K

        def body(i, _):
            pltpu.sync_copy(dst_hbm.at[pl.ds(base + i * K, K)], idx_v)
            pltpu.sync_copy(ones_v, deg_sh.at[idx_v], add=True)
            return 0

        lax.fori_loop(0, chunks, body, 0)
        plsc.subcore_barrier()
        pltpu.sync_copy(
            deg_sh.at[pl.ds(s * rt, rt)],
            degpart_hbm.at[pl.ds(c * npad + s * rt, rt)],
        )

    return deg_k(dst_e)


def _linear_scale_call(x, w_mat, degpart, npad):
    n, _ = x.shape
    d_out = w_mat.shape[0]
    rb = 512
    grid = npad // rb
    dp3 = degpart.reshape(NC, npad, DEGW)

    def body(x_ref, w_ref, dp_ref, g_ref):
        i = pl.program_id(0)
        h = lax.dot_general(
            x_ref[...], w_ref[...], (((1,), (1,)), ((), ())),
            preferred_element_type=jnp.float32,
        )
        deg = dp_ref[0, :, 0:1] + dp_ref[1, :, 0:1]
        dis = lax.rsqrt(deg)
        rows = i * rb + lax.broadcasted_iota(jnp.int32, (rb, 1), 0)
        g_ref[...] = jnp.where(rows < n, h * dis, 0.0)

    return pl.pallas_call(
        body,
        grid=(grid,),
        in_specs=[
            pl.BlockSpec((rb, x.shape[1]), lambda i: (i, 0)),
            pl.BlockSpec((d_out, w_mat.shape[1]), lambda i: (0, 0)),
            pl.BlockSpec((NC, rb, DEGW), lambda i: (0, i, 0)),
        ],
        out_specs=pl.BlockSpec((rb, d_out), lambda i: (i, 0)),
        out_shape=jax.ShapeDtypeStruct((npad, d_out), jnp.float32),
    )(x, w_mat, dp3)


def _scatter_call(g, src_e, dst_e, npad, chunks):
    d = g.shape[1]
    rt = npad // NS

    @functools.partial(
        pl.kernel,
        out_type=jax.ShapeDtypeStruct((NC * npad, d), jnp.float32),
        mesh=_mesh(),
        scratch_types=[
            pltpu.VMEM_SHARED((npad, d), jnp.float32),
            pltpu.VMEM((K,), jnp.int32),
            pltpu.VMEM((K,), jnp.int32),
            pltpu.VMEM((K, d), jnp.float32),
            pltpu.SemaphoreType.DMA,
        ],
    )
    def acc_k(g_hbm, src_hbm, dst_hbm, accpart_hbm, acc_sh, sidx_v, didx_v,
              rows_v, sem):
        c = lax.axis_index("c")
        s = lax.axis_index("s")
        w = c * NS + s
        _fill_rows(rows_v, K, d // 16, 0.0)
        for j in range(rt // K):
            pltpu.sync_copy(rows_v, acc_sh.at[pl.ds(s * rt + j * K, K)])
        plsc.subcore_barrier()

        base = w * chunks * K

        def body(i, _):
            pltpu.sync_copy(src_hbm.at[pl.ds(base + i * K, K)], sidx_v)
            pltpu.sync_copy(dst_hbm.at[pl.ds(base + i * K, K)], didx_v)
            pltpu.async_copy(g_hbm.at[sidx_v], rows_v, sem).wait()
            pltpu.sync_copy(rows_v, acc_sh.at[didx_v], add=True)
            return 0

        lax.fori_loop(0, chunks, body, 0)
        plsc.subcore_barrier()
        pltpu.sync_copy(
            acc_sh.at[pl.ds(s * rt, rt)],
            accpart_hbm.at[pl.ds(c * npad + s * rt, rt)],
        )

    return acc_k(g, src_e, dst_e)


def _combine_call(accpart, degpart, b, n, npad):
    d = accpart.shape[1]
    rb = 1000
    grid = -(-n // rb)
    ap3 = accpart.reshape(NC, npad, d)
    dp3 = degpart.reshape(NC, npad, DEGW)
    b2 = b.reshape(1, d)

    def body(ap_ref, dp_ref, b_ref, o_ref):
        acc = ap_ref[0] + ap_ref[1]
        deg = dp_ref[0, :, 0:1] + dp_ref[1, :, 0:1]
        o_ref[...] = acc * lax.rsqrt(deg) + b_ref[...]

    return pl.pallas_call(
        body,
        grid=(grid,),
        in_specs=[
            pl.BlockSpec((NC, rb, d), lambda i: (0, i, 0)),
            pl.BlockSpec((NC, rb, DEGW), lambda i: (0, i, 0)),
            pl.BlockSpec((1, d), lambda i: (0, 0)),
        ],
        out_specs=pl.BlockSpec((rb, d), lambda i: (i, 0)),
        out_shape=jax.ShapeDtypeStruct((n, d), jnp.float32),
    )(ap3, dp3, b2)


def kernel(x, edge_index, W, b):
    n = x.shape[0]
    e = edge_index.shape[1]
    src = edge_index[0].astype(jnp.int32)
    dst = edge_index[1].astype(jnp.int32)
    loop = jnp.arange(n, dtype=jnp.int32)
    src_e = jnp.concatenate([src, loop])
    dst_e = jnp.concatenate([dst, loop])
    e_ext = e + n
    chunks = -(-e_ext // (NW * K))
    e_pad = chunks * NW * K
    src_e = jnp.pad(src_e, (0, e_pad - e_ext), constant_values=n)
    dst_e = jnp.pad(dst_e, (0, e_pad - e_ext), constant_values=n)
    npad = -(-(n + 1) // (NS * K)) * (NS * K)

    degpart = _deg_call(dst_e, npad, chunks)
    g = _linear_scale_call(x, W, degpart, npad)
    accpart = _scatter_call(g, src_e, dst_e, npad, chunks)
    return _combine_call(accpart, degpart, b, n, npad)


# speedup vs baseline: 18.4974x; 1.0971x over previous
import functools

import jax
import jax.numpy as jnp
from jax import lax
from jax.experimental import pallas as pl
from jax.experimental.pallas import tpu as pltpu
from jax.experimental.pallas import tpu_sc as plsc

NC = 2
NS = 16
NW = NC * NS
K = 128
DEGW = 16
NBUF = 4


def _mesh():
    return plsc.VectorSubcoreMesh(
        core_axis_name="c", subcore_axis_name="s", num_cores=NC, num_subcores=NS
    )


def _fill_rows(ref, n_rows, n_lane_groups, value):
    vec = jnp.full((16,), value, jnp.float32)

    def body(i, _):
        for j in range(n_lane_groups):
            ref[i, pl.ds(16 * j, 16)] = vec
        return 0

    lax.fori_loop(0, n_rows, body, 0)


def _deg_call(dst_e, npad, chunks):
    rt = npad // NS

    @functools.partial(
        pl.kernel,
        out_type=jax.ShapeDtypeStruct((NC * npad, DEGW), jnp.float32),
        mesh=_mesh(),
        scratch_types=[
            pltpu.VMEM_SHARED((npad, DEGW), jnp.float32),
            pltpu.VMEM((K,), jnp.int32),
            pltpu.VMEM((K, DEGW), jnp.float32),
            pltpu.VMEM((rt, DEGW), jnp.float32),
        ],
    )
    def deg_k(dst_hbm, degpart_hbm, deg_sh, idx_v, ones_v, z_v):
        c = lax.axis_index("c")
        s = lax.axis_index("s")
        w = c * NS + s
        _fill_rows(z_v, rt, DEGW // 16, 0.0)
        pltpu.sync_copy(z_v, deg_sh.at[pl.ds(s * rt, rt)])
        _fill_rows(ones_v, K, DEGW // 16, 1.0)
        plsc.subcore_barrier()

        base = w * chunks * K

        def body(i, _):
            pltpu.sync_copy(dst_hbm.at[pl.ds(base + i * K, K)], idx_v)
            pltpu.sync_copy(ones_v, deg_sh.at[idx_v], add=True)
            return 0

        lax.fori_loop(0, chunks, body, 0)
        plsc.subcore_barrier()
        pltpu.sync_copy(
            deg_sh.at[pl.ds(s * rt, rt)],
            degpart_hbm.at[pl.ds(c * npad + s * rt, rt)],
        )

    return deg_k(dst_e)


def _linear_scale_call(x, w_mat, degpart, npad):
    n, _ = x.shape
    d_out = w_mat.shape[0]
    dh = d_out // NC
    rb = 512
    grid = npad // rb
    dp3 = degpart.reshape(NC, npad, DEGW)

    def body(x_ref, w_ref, dp_ref, g_ref):
        i = pl.program_id(0)
        h = lax.dot_general(
            x_ref[...], w_ref[...], (((1,), (1,)), ((), ())),
            preferred_element_type=jnp.float32,
        )
        deg = dp_ref[0, :, 0:1] + dp_ref[1, :, 0:1]
        dis = lax.rsqrt(deg)
        rows = i * rb + lax.broadcasted_iota(jnp.int32, (rb, 1), 0)
        g = jnp.where(rows < n, h * dis, 0.0)
        for c in range(NC):
            g_ref[c] = g[:, c * dh:(c + 1) * dh]

    return pl.pallas_call(
        body,
        grid=(grid,),
        in_specs=[
            pl.BlockSpec((rb, x.shape[1]), lambda i: (i, 0)),
            pl.BlockSpec((d_out, w_mat.shape[1]), lambda i: (0, 0)),
            pl.BlockSpec((NC, rb, DEGW), lambda i: (0, i, 0)),
        ],
        out_specs=pl.BlockSpec((NC, rb, dh), lambda i: (0, i, 0)),
        out_shape=jax.ShapeDtypeStruct((NC, npad, dh), jnp.float32),
    )(x, w_mat, dp3)


def _scatter_call(g2, src3a, src3b, dst3, npad, per_tile_chunks):
    dh = g2.shape[1]
    rt = npad // NS
    pc = per_tile_chunks

    @functools.partial(
        pl.kernel,
        out_type=jax.ShapeDtypeStruct((NC * npad, dh), jnp.float32),
        mesh=_mesh(),
        scratch_types=(
            [
                pltpu.VMEM_SHARED((npad, dh), jnp.float32),
                pltpu.VMEM((pc, 1, K), jnp.int32),
                pltpu.VMEM((pc, 1, K), jnp.int32),
            ]
            + [pltpu.VMEM((K, dh), jnp.float32) for _ in range(NBUF)]
            + [pltpu.SemaphoreType.DMA for _ in range(NBUF)]
        ),
        compiler_params=pltpu.CompilerParams(use_tc_tiling_on_sc=False),
    )
    def acc_k(g_hbm, sa_hbm, sb_hbm, d_hbm, accpart_hbm, acc_sh, sidx_v,
              didx_v, *bufs):
        rows_v = bufs[:NBUF]
        sems = bufs[NBUF:]
        c = lax.axis_index("c")
        s = lax.axis_index("s")
        _fill_rows(rows_v[0], K, dh // 16, 0.0)
        for j in range(rt // K):
            pltpu.sync_copy(rows_v[0], acc_sh.at[pl.ds(s * rt + j * K, K)])
        plsc.subcore_barrier()

        base = s * pc
        @pl.when(c == 0)
        def _():
            pltpu.sync_copy(sa_hbm.at[pl.ds(base, pc)], sidx_v)

        @pl.when(c != 0)
        def _():
            pltpu.sync_copy(sb_hbm.at[pl.ds(base, pc)], sidx_v)

        pltpu.sync_copy(d_hbm.at[pl.ds(base, pc)], didx_v)

        for r in range(NBUF):
            pltpu.async_copy(g_hbm.at[sidx_v.at[r, 0]], rows_v[r], sems[r])

        def body(i, _):
            for r in range(NBUF):
                ch = i * NBUF + r
                pltpu.make_async_copy(
                    g_hbm.at[sidx_v.at[ch, 0]], rows_v[r], sems[r]
                ).wait()
                pltpu.sync_copy(rows_v[r], acc_sh.at[didx_v.at[ch, 0]],
                                add=True)

                @pl.when(ch + NBUF < pc)
                def _():
                    pltpu.async_copy(g_hbm.at[sidx_v.at[ch + NBUF, 0]],
                                     rows_v[r], sems[r])
            return 0

        lax.fori_loop(0, pc // NBUF, body, 0)
        plsc.subcore_barrier()
        pltpu.sync_copy(
            acc_sh.at[pl.ds(s * rt, rt)],
            accpart_hbm.at[pl.ds(c * npad + s * rt, rt)],
        )

    return acc_k(g2, src3a, src3b, dst3)


def _combine_call(accpart, degpart, b, n, npad):
    dh = accpart.shape[1]
    d = NC * dh
    rb = 1000
    grid = -(-n // rb)
    ap3 = accpart.reshape(NC, npad, dh)
    dp3 = degpart.reshape(NC, npad, DEGW)
    b2 = b.reshape(1, d)

    def body(ap_ref, dp_ref, b_ref, o_ref):
        deg = dp_ref[0, :, 0:1] + dp_ref[1, :, 0:1]
        dis = lax.rsqrt(deg)
        for c in range(NC):
            o_ref[:, c * dh:(c + 1) * dh] = (
                ap_ref[c] * dis + b_ref[:, c * dh:(c + 1) * dh]
            )

    return pl.pallas_call(
        body,
        grid=(grid,),
        in_specs=[
            pl.BlockSpec((NC, rb, dh), lambda i: (0, i, 0)),
            pl.BlockSpec((NC, rb, DEGW), lambda i: (0, i, 0)),
            pl.BlockSpec((1, d), lambda i: (0, 0)),
        ],
        out_specs=pl.BlockSpec((rb, d), lambda i: (i, 0)),
        out_shape=jax.ShapeDtypeStruct((n, d), jnp.float32),
    )(ap3, dp3, b2)


def kernel(x, edge_index, W, b):
    n = x.shape[0]
    e = edge_index.shape[1]
    src = edge_index[0].astype(jnp.int32)
    dst = edge_index[1].astype(jnp.int32)
    loop = jnp.arange(n, dtype=jnp.int32)
    src_e = jnp.concatenate([src, loop])
    dst_e = jnp.concatenate([dst, loop])
    e_ext = e + n
    tchunks = -(-(-(-e_ext // K)) // (NS * NBUF)) * (NS * NBUF)
    e_pad = tchunks * K
    src_e = jnp.pad(src_e, (0, e_pad - e_ext), constant_values=n)
    dst_e = jnp.pad(dst_e, (0, e_pad - e_ext), constant_values=n)
    npad = -(-(n + 1) // (NS * K)) * (NS * K)
    src3a = src_e.reshape(-1, 1, K)
    src3b = src3a + npad
    dst3 = dst_e.reshape(-1, 1, K)

    degpart = _deg_call(dst_e, npad, tchunks // NW)
    g2 = _linear_scale_call(x, W, degpart, npad)
    accpart = _scatter_call(
        g2.reshape(NC * npad, -1), src3a, src3b, dst3, npad, tchunks // NS
    )
    return _combine_call(accpart, degpart, b, n, npad)
